# baseline (speedup 1.0000x reference)
import jax
import jax.numpy as jnp
from jax import lax
from jax.experimental import pallas as pl
from jax.experimental.pallas import tpu as pltpu
from jax.experimental.pallas import tpu_sc as plsc

_P = 8
_MASK_RATIO = 0.75
_CC = 64
_ROWS, _LANES = 224, 224
_NP = 784
_JPW = 112
_LN = 16


def _sc_mask_kernel(noise_hbm, mask_hbm, noise_v, mask_v):
    b_total = noise_hbm.shape[0] // _NP
    num_keep = int(_NP * (1.0 - _MASK_RATIO))
    nworkers = b_total * (_NP // _JPW)

    wid = lax.axis_index("s") * 2 + lax.axis_index("c")

    @pl.when(wid < nworkers)
    def _():
        b = wid // (_NP // _JPW)
        part = wid % (_NP // _JPW)
        pltpu.sync_copy(noise_hbm.at[pl.ds(b * _NP, _NP)],
                        noise_v.at[pl.ds(0, _NP)])

        lane = lax.broadcasted_iota(jnp.int32, (_LN,), 0)
        nchunk = _JPW // _LN
        j0s = [part * _JPW + jc * _LN for jc in range(nchunk)]
        njvs = [noise_v[pl.ds(j0, _LN)] for j0 in j0s]
        jidxs = [lane + j0 for j0 in j0s]
        one = jnp.ones((_LN,), jnp.float32)
        zero = jnp.zeros((_LN,), jnp.float32)

        def body(k, cnts):
            nkb = jnp.full((_LN,), noise_v[pl.ds(k, _LN)][0])
            out = []
            for jc in range(nchunk):
                lt = nkb < njvs[jc]
                tie = (nkb == njvs[jc]) & (k < jidxs[jc])
                out.append(cnts[jc] + jnp.where(lt | tie, one, zero))
            return tuple(out)

        ranks = lax.fori_loop(0, _NP, body, tuple(zero for _ in range(nchunk)),
                              unroll=8)
        for jc in range(nchunk):
            mask_v[pl.ds(jc * _LN, _LN)] = jnp.where(
                ranks[jc] < float(num_keep), one, zero)

        pltpu.sync_copy(mask_v,
                        mask_hbm.at[pl.ds(b * _NP + part * _JPW, _JPW)])


def _sc_mask(noise):
    b = noise.shape[0]
    mesh = plsc.VectorSubcoreMesh(core_axis_name="c", subcore_axis_name="s")
    flat = pl.kernel(
        _sc_mask_kernel,
        mesh=mesh,
        out_type=jax.ShapeDtypeStruct((b * _NP,), jnp.float32),
        scratch_types=[
            pltpu.VMEM((_NP + _LN,), jnp.float32),
            pltpu.VMEM((_JPW,), jnp.float32),
        ],
    )(noise.reshape(b * _NP))
    return flat.reshape(b, _NP)


def _tc_multiply_kernel(mask_ref, x_ref, out_ref, spat_ref):
    nc = pl.program_id(1)
    hp = 224 // _P

    @pl.when(nc == 0)
    def _expand_mask():
        m = mask_ref[0]
        a_s = lax.broadcasted_iota(jnp.int32, (_ROWS, _NP), 0)
        a_p = lax.broadcasted_iota(jnp.int32, (_ROWS, _NP), 1)
        a_sel = ((a_p // hp) == (a_s // _P)).astype(jnp.float32)
        b_p = lax.broadcasted_iota(jnp.int32, (_NP, _LANES), 0)
        b_l = lax.broadcasted_iota(jnp.int32, (_NP, _LANES), 1)
        b_sel = ((b_p % hp) == (b_l // _P)).astype(jnp.float32)
        spat_ref[...] = jnp.dot(a_sel, m * b_sel,
                                preferred_element_type=jnp.float32)

    out_ref[...] = x_ref[...] * spat_ref[...][None, None, :, :]


def kernel(x, noise):
    b, c, h_full, w_full = x.shape
    num_patches = noise.shape[1]
    nc = c // _CC

    mask = _sc_mask(noise)
    mask3 = mask.reshape(b, num_patches, 1)

    x_img = pl.pallas_call(
        _tc_multiply_kernel,
        grid=(b, nc),
        in_specs=[
            pl.BlockSpec((1, num_patches, 1), lambda i, j: (i, 0, 0)),
            pl.BlockSpec((1, _CC, _ROWS, _LANES), lambda i, j: (i, j, 0, 0)),
        ],
        out_specs=pl.BlockSpec((1, _CC, _ROWS, _LANES),
                               lambda i, j: (i, j, 0, 0)),
        out_shape=jax.ShapeDtypeStruct((b, c, _ROWS, _LANES), x.dtype),
        scratch_shapes=[pltpu.VMEM((_ROWS, _LANES), jnp.float32)],
        compiler_params=pltpu.CompilerParams(
            dimension_semantics=("parallel", "arbitrary"),
        ),
    )(mask3, x)

    return (x_img, mask)


# speedup vs baseline: 1.5903x; 1.5903x over previous
import jax
import jax.numpy as jnp
from jax import lax
from jax.experimental import pallas as pl
from jax.experimental.pallas import tpu as pltpu
from jax.experimental.pallas import tpu_sc as plsc

_P = 8
_MASK_RATIO = 0.75
_CC = 64
_ROWS, _LANES = 224, 224
_NP = 784
_JPW = 112
_LN = 16


def _sc_mask_kernel(noise_hbm, mask_hbm, noise_v, mask_v):
    b_total = noise_hbm.shape[0] // _NP
    num_keep = int(_NP * (1.0 - _MASK_RATIO))
    nworkers = b_total * (_NP // _JPW)

    wid = lax.axis_index("s") * 2 + lax.axis_index("c")

    @pl.when(wid < nworkers)
    def _():
        b = wid // (_NP // _JPW)
        part = wid % (_NP // _JPW)
        pltpu.sync_copy(noise_hbm.at[pl.ds(b * _NP, _NP)],
                        noise_v.at[pl.ds(0, _NP)])

        lane = lax.broadcasted_iota(jnp.int32, (_LN,), 0)
        one = jnp.ones((_LN,), jnp.float32)
        zero = jnp.zeros((_LN,), jnp.float32)
        for jc in range(_JPW // _LN):
            j0 = part * _JPW + jc * _LN
            njv = noise_v[pl.ds(j0, _LN)]
            jidx = lane + j0

            def body(k, cnt):
                nkb = jnp.full((_LN,), noise_v[pl.ds(k, _LN)][0])
                lt = nkb < njv
                tie = (nkb == njv) & (k < jidx)
                return cnt + jnp.where(lt | tie, one, zero)

            rank = lax.fori_loop(0, _NP, body, zero, unroll=16)
            mask_v[pl.ds(jc * _LN, _LN)] = jnp.where(
                rank < float(num_keep), one, zero)

        pltpu.sync_copy(mask_v,
                        mask_hbm.at[pl.ds(b * _NP + part * _JPW, _JPW)])


def _sc_mask(noise):
    b = noise.shape[0]
    mesh = plsc.VectorSubcoreMesh(core_axis_name="c", subcore_axis_name="s")
    flat = pl.kernel(
        _sc_mask_kernel,
        mesh=mesh,
        out_type=jax.ShapeDtypeStruct((b * _NP,), jnp.float32),
        scratch_types=[
            pltpu.VMEM((_NP + _LN,), jnp.float32),
            pltpu.VMEM((_JPW,), jnp.float32),
        ],
    )(noise.reshape(b * _NP))
    return flat.reshape(b, _NP)


def _tc_multiply_kernel(mask_ref, x_ref, out_ref, spat_ref):
    nc = pl.program_id(1)
    hp = 224 // _P

    @pl.when(nc == 0)
    def _expand_mask():
        m = mask_ref[0]
        a_s = lax.broadcasted_iota(jnp.int32, (_ROWS, _NP), 0)
        a_p = lax.broadcasted_iota(jnp.int32, (_ROWS, _NP), 1)
        a_sel = ((a_p // hp) == (a_s // _P)).astype(jnp.float32)
        b_p = lax.broadcasted_iota(jnp.int32, (_NP, _LANES), 0)
        b_l = lax.broadcasted_iota(jnp.int32, (_NP, _LANES), 1)
        b_sel = ((b_p % hp) == (b_l // _P)).astype(jnp.float32)
        spat_ref[...] = jnp.dot(a_sel, m * b_sel,
                                preferred_element_type=jnp.float32)

    out_ref[...] = x_ref[...] * spat_ref[...][None, None, :, :]


def kernel(x, noise):
    b, c, h_full, w_full = x.shape
    num_patches = noise.shape[1]
    nc = c // _CC

    mask = _sc_mask(noise)
    mask3 = mask.reshape(b, num_patches, 1)

    x_img = pl.pallas_call(
        _tc_multiply_kernel,
        grid=(b, nc),
        in_specs=[
            pl.BlockSpec((1, num_patches, 1), lambda i, j: (i, 0, 0)),
            pl.BlockSpec((1, _CC, _ROWS, _LANES), lambda i, j: (i, j, 0, 0)),
        ],
        out_specs=pl.BlockSpec((1, _CC, _ROWS, _LANES),
                               lambda i, j: (i, j, 0, 0)),
        out_shape=jax.ShapeDtypeStruct((b, c, _ROWS, _LANES), x.dtype),
        scratch_shapes=[pltpu.VMEM((_ROWS, _LANES), jnp.float32)],
        compiler_params=pltpu.CompilerParams(
            dimension_semantics=("parallel", "arbitrary"),
        ),
    )(mask3, x)

    return (x_img, mask)
